# baseline (speedup 1.0000x reference)
import functools

import jax
import jax.numpy as jnp
from jax import lax
from jax.experimental import pallas as pl
from jax.experimental.pallas import tpu as pltpu
from jax.experimental.pallas import tpu_sc as plsc

_PERM = (
    1, 3, 5, 0, 2, 6, 7, 4,
    10, 8, 12, 13, 15, 14, 11, 9,
    21, 23, 16, 17, 20, 18, 22, 19,
    28, 29, 27, 26, 31, 30, 24, 25,
    38, 37, 33, 35, 34, 39, 36, 32,
    43, 41, 47, 42, 44, 40, 45, 46,
    49, 55, 54, 48, 53, 51, 52, 50,
    61, 56, 58, 63, 57, 59, 60, 62,
)

_B, _C, _T = 128, 64, 4000
_NW = 32
_CH_PW = _C // _NW
_WORDS = tuple(_PERM[2 * w] | (_PERM[2 * w + 1] << 8) for w in range(_NW))

_CHUNK = 200
_NCHUNK = _T // _CHUNK
_NBUF = 4


def _make_sc_permute():
    mesh = plsc.VectorSubcoreMesh(core_axis_name="c", subcore_axis_name="s")

    @functools.partial(
        pl.kernel,
        mesh=mesh,
        out_type=jax.ShapeDtypeStruct((_C, 1, _T, _B), jnp.float32),
        scratch_types=(
            [pltpu.VMEM((1, 1, _CHUNK, _B), jnp.float32)] * 2
            + [pltpu.VMEM_SHARED((16, 2, _CHUNK, _B), jnp.float32)]
            + [pltpu.SemaphoreType.DMA] * 8
        ),
    )
    def sc_permute(in_hbm, out_hbm, vb0, vb1, shared, *sems):
        sid = lax.axis_index("s")
        wid = sid * 2 + lax.axis_index("c")
        word = jnp.int32(0)
        for w in range(_NW):
            word = jnp.where(wid == w, jnp.int32(_WORDS[w]), word)
        srcs = (word & 0xFF, word >> 8)
        dsts = (wid * _CH_PW, wid * _CH_PW + 1)
        path_bufs = (
            (vb0, vb1),
            tuple(shared.at[pl.ds(sid, 1), pl.ds(p, 1), :, :] for p in range(2)),
        )
        rsems = (sems[0:2], sems[2:4])
        wsems = (sems[4:6], sems[6:8])

        def read(ch, k):
            p = k % 2
            return pltpu.async_copy(
                in_hbm.at[pl.ds(srcs[ch], 1), :, pl.ds(k * _CHUNK, _CHUNK), :],
                path_bufs[ch][p],
                rsems[ch][p],
            )

        def write(ch, k):
            p = k % 2
            return pltpu.async_copy(
                path_bufs[ch][p],
                out_hbm.at[pl.ds(dsts[ch], 1), :, pl.ds(k * _CHUNK, _CHUNK), :],
                wsems[ch][p],
            )

        pending_reads = [[None, None], [None, None]]
        pending_writes = [[None, None], [None, None]]
        lag = 1
        for k in range(_NCHUNK + lag):
            for ch in range(2):
                if k < _NCHUNK:
                    p = k % 2
                    if pending_writes[ch][p] is not None:
                        pending_writes[ch][p].wait()
                    pending_reads[ch][p] = read(ch, k)
                if k >= lag:
                    j = k - lag
                    q = j % 2
                    pending_reads[ch][q].wait()
                    pending_writes[ch][q] = write(ch, j)
        for ch in range(2):
            for j in (_NCHUNK - 2, _NCHUNK - 1):
                pending_writes[ch][j % 2].wait()

    return sc_permute


def kernel(data_tensor, domain_labels, aux_labels):
    del domain_labels, aux_labels
    x = jnp.transpose(data_tensor, (1, 2, 3, 0))
    y = _make_sc_permute()(x)
    return jnp.transpose(y, (3, 0, 1, 2))


# speedup vs baseline: 1.0117x; 1.0117x over previous
import functools

import jax
import jax.numpy as jnp
from jax import lax
from jax.experimental import pallas as pl
from jax.experimental.pallas import tpu as pltpu
from jax.experimental.pallas import tpu_sc as plsc

_PERM = (
    1, 3, 5, 0, 2, 6, 7, 4,
    10, 8, 12, 13, 15, 14, 11, 9,
    21, 23, 16, 17, 20, 18, 22, 19,
    28, 29, 27, 26, 31, 30, 24, 25,
    38, 37, 33, 35, 34, 39, 36, 32,
    43, 41, 47, 42, 44, 40, 45, 46,
    49, 55, 54, 48, 53, 51, 52, 50,
    61, 56, 58, 63, 57, 59, 60, 62,
)

_B, _C, _T = 128, 64, 4000
_NW = 32
_CH_PW = _C // _NW
_WORDS = tuple(_PERM[2 * w] | (_PERM[2 * w + 1] << 8) for w in range(_NW))

_CHUNK = 200
_NCHUNK = _T // _CHUNK
_NBUF = 4


def _make_sc_permute():
    mesh = plsc.VectorSubcoreMesh(core_axis_name="c", subcore_axis_name="s")

    @functools.partial(
        pl.kernel,
        mesh=mesh,
        out_type=jax.ShapeDtypeStruct((_C, 1, _T, _B), jnp.float32),
        scratch_types=(
            [pltpu.VMEM_SHARED((16, _NBUF, _CHUNK, _B), jnp.float32)]
            + [pltpu.SemaphoreType.DMA] * (2 * _NBUF)
        ),
    )
    def sc_permute(in_hbm, out_hbm, shared, *sems):
        sid = lax.axis_index("s")
        bufs = [
            shared.at[pl.ds(sid, 1), pl.ds(p, 1), :, :] for p in range(_NBUF)
        ]
        rsems = sems[:_NBUF]
        wsems = sems[_NBUF:]
        wid = sid * 2 + lax.axis_index("c")
        word = jnp.int32(0)
        for w in range(_NW):
            word = jnp.where(wid == w, jnp.int32(_WORDS[w]), word)
        srcs = (word & 0xFF, word >> 8)
        dsts = (wid * _CH_PW, wid * _CH_PW + 1)

        steps = [(ch, k) for ch in range(_CH_PW) for k in range(_NCHUNK)]
        n = len(steps)

        def read(i):
            ch, k = steps[i]
            p = i % _NBUF
            return pltpu.async_copy(
                in_hbm.at[pl.ds(srcs[ch], 1), :, pl.ds(k * _CHUNK, _CHUNK), :],
                bufs[p],
                rsems[p],
            )

        def write(i):
            ch, k = steps[i]
            p = i % _NBUF
            return pltpu.async_copy(
                bufs[p],
                out_hbm.at[pl.ds(dsts[ch], 1), :, pl.ds(k * _CHUNK, _CHUNK), :],
                wsems[p],
            )

        pending_reads = [None] * _NBUF
        pending_writes = [None] * _NBUF
        lag = 1
        for i in range(n + lag):
            if i < n:
                p = i % _NBUF
                if pending_writes[p] is not None:
                    pending_writes[p].wait()
                pending_reads[p] = read(i)
            if i >= lag:
                j = i - lag
                q = j % _NBUF
                pending_reads[q].wait()
                pending_writes[q] = write(j)
        for j in range(n - _NBUF, n):
            pending_writes[j % _NBUF].wait()

    return sc_permute


def kernel(data_tensor, domain_labels, aux_labels):
    del domain_labels, aux_labels
    x = jnp.transpose(data_tensor, (1, 2, 3, 0))
    y = _make_sc_permute()(x)
    return jnp.transpose(y, (3, 0, 1, 2))
